# Initial kernel scaffold; baseline (speedup 1.0000x reference)
import jax
import jax.numpy as jnp
from jax.experimental import pallas as pl


def kernel(x, edge_index, W, b):
    raise NotImplementedError("write your pallas kernel here")


# speedup vs baseline: 11.1177x; 11.1177x over previous
import functools

import jax
import jax.numpy as jnp
from jax import lax
from jax.experimental import pallas as pl
from jax.experimental.pallas import tpu as pltpu
from jax.experimental.pallas import tpu_sc as plsc

NC = 2
NS = 16
NW = NC * NS
CH = 128
HW = 16


def _sc_mesh():
    return plsc.VectorSubcoreMesh(
        core_axis_name="c", subcore_axis_name="s", num_cores=NC, num_subcores=NS
    )


def _make_hist(n_pad, e_pad):
    iters = e_pad // (NW * CH)
    rpt = n_pad // NS
    n_dma = rpt // 125 if rpt % 125 == 0 else 1
    blk = rpt // n_dma

    def body(dst_hbm, out_hbm, acc, idx_v, ones_v, stage_v, gsem):
        c = lax.axis_index("c")
        s = lax.axis_index("s")
        g = c * NS + s
        ones16 = jnp.full((16,), 1.0, jnp.float32)
        zeros16 = jnp.zeros((16,), jnp.float32)

        def fill_ones(i, _):
            ones_v[i, :] = ones16
            return 0

        lax.fori_loop(0, CH, fill_ones, 0)

        def fill_zero(i, _):
            stage_v[i, :] = zeros16
            return 0

        lax.fori_loop(0, blk, fill_zero, 0)

        base = s * rpt
        for k in range(n_dma):
            pltpu.sync_copy(stage_v, acc.at[pl.ds(base + k * blk, blk)])
        plsc.subcore_barrier()

        ebase = g * iters * CH

        def chunk(i, _):
            pltpu.sync_copy(dst_hbm.at[pl.ds(ebase + i * CH, CH)], idx_v)
            pltpu.sync_copy(ones_v, acc.at[idx_v], add=True)
            return 0

        lax.fori_loop(0, iters, chunk, 0)
        plsc.subcore_barrier()

        for k in range(n_dma):
            pltpu.sync_copy(acc.at[pl.ds(base + k * blk, blk)], stage_v)
            pltpu.sync_copy(stage_v, out_hbm.at[c, pl.ds(base + k * blk, blk)])

    return pl.kernel(
        body,
        out_type=jax.ShapeDtypeStruct((NC, n_pad, HW), jnp.float32),
        mesh=_sc_mesh(),
        compiler_params=pltpu.CompilerParams(use_tc_tiling_on_sc=False),
        scratch_types=[
            pltpu.VMEM_SHARED((n_pad, HW), jnp.float32),
            pltpu.VMEM((CH,), jnp.int32),
            pltpu.VMEM((CH, HW), jnp.float32),
            pltpu.VMEM((blk, HW), jnp.float32),
            pltpu.SemaphoreType.DMA,
        ],
    )


def _make_hop(n_pad, e_pad, d):
    iters = e_pad // (NW * CH)
    rpt = n_pad // NS
    blk = 104
    n_full = rpt // blk
    tail = rpt - n_full * blk
    assert tail % 8 == 0

    def body(src_hbm, dst_hbm, y_hbm, out_hbm, acc, idx_s, idx_d, rows_v,
             stage_v, gsem):
        c = lax.axis_index("c")
        s = lax.axis_index("s")
        g = c * NS + s
        zeros16 = jnp.zeros((16,), jnp.float32)

        def fill_zero(i, _):
            def fill_col(j, _):
                stage_v[i, pl.ds(j * 16, 16)] = zeros16
                return 0

            lax.fori_loop(0, d // 16, fill_col, 0)
            return 0

        lax.fori_loop(0, blk, fill_zero, 0)

        base = s * rpt

        def zero_blk(k, _):
            pltpu.sync_copy(stage_v, acc.at[pl.ds(base + k * blk, blk)])
            return 0

        lax.fori_loop(0, n_full, zero_blk, 0)
        if tail:
            pltpu.sync_copy(
                stage_v.at[pl.ds(0, tail)],
                acc.at[pl.ds(base + n_full * blk, tail)],
            )
        plsc.subcore_barrier()

        ebase = g * iters * CH

        def chunk(i, _):
            off = ebase + i * CH
            pltpu.sync_copy(src_hbm.at[pl.ds(off, CH)], idx_s)
            pltpu.sync_copy(dst_hbm.at[pl.ds(off, CH)], idx_d)
            pltpu.async_copy(y_hbm.at[idx_s], rows_v, gsem).wait()
            pltpu.sync_copy(rows_v, acc.at[idx_d], add=True)
            return 0

        lax.fori_loop(0, iters, chunk, 0)
        plsc.subcore_barrier()

        def read_blk(k, _):
            pltpu.sync_copy(acc.at[pl.ds(base + k * blk, blk)], stage_v)
            pltpu.sync_copy(stage_v, out_hbm.at[c, pl.ds(base + k * blk, blk)])
            return 0

        lax.fori_loop(0, n_full, read_blk, 0)
        if tail:
            pltpu.sync_copy(
                acc.at[pl.ds(base + n_full * blk, tail)],
                stage_v.at[pl.ds(0, tail)],
            )
            pltpu.sync_copy(
                stage_v.at[pl.ds(0, tail)],
                out_hbm.at[c, pl.ds(base + n_full * blk, tail)],
            )

    return pl.kernel(
        body,
        out_type=jax.ShapeDtypeStruct((NC, n_pad, d), jnp.float32),
        mesh=_sc_mesh(),
        scratch_types=[
            pltpu.VMEM_SHARED((n_pad, d), jnp.float32),
            pltpu.VMEM((CH,), jnp.int32),
            pltpu.VMEM((CH,), jnp.int32),
            pltpu.VMEM((CH, d), jnp.float32),
            pltpu.VMEM((blk, d), jnp.float32),
            pltpu.SemaphoreType.DMA,
        ],
    )


def _tc_scale0(hist_ref, x_ref, dinv_ref, y1_ref):
    deg = 1.0 + hist_ref[0, :, 0:1] + hist_ref[1, :, 0:1]
    dinv = lax.rsqrt(deg)
    dinv_ref[...] = dinv
    y1_ref[...] = x_ref[...] * dinv


def _tc_scale1(part_ref, y1_ref, dinv_ref, y2_ref):
    dv = dinv_ref[...]
    y2_ref[...] = (part_ref[0] + part_ref[1] + y1_ref[...]) * (dv * dv)


def _tc_final(part_ref, y2_ref, dinv_ref, wt_ref, b_ref, o_ref):
    h2 = (part_ref[0] + part_ref[1] + y2_ref[...]) * dinv_ref[...]
    o_ref[...] = (
        jnp.dot(h2, wt_ref[...], preferred_element_type=jnp.float32) + b_ref[...]
    )


def kernel(x, edge_index, W, b):
    n, d = x.shape
    e = edge_index.shape[1]
    assert n % NS == 0 and d % 16 == 0

    e_pad = -(-e // (NW * CH)) * (NW * CH)
    n_pad = -(-(n + 1) // (NS * 8)) * (NS * 8)
    src = edge_index[0]
    dst = edge_index[1]
    if e_pad != e:
        pad = e_pad - e
        src = jnp.concatenate([src, jnp.zeros((pad,), jnp.int32)])
        dst = jnp.concatenate([dst, jnp.full((pad,), n, jnp.int32)])

    hist = _make_hist(n_pad, e_pad)(dst)
    hop = _make_hop(n_pad, e_pad, d)

    dinv, y1 = pl.pallas_call(
        _tc_scale0,
        out_shape=[
            jax.ShapeDtypeStruct((n, 1), jnp.float32),
            jax.ShapeDtypeStruct((n, d), jnp.float32),
        ],
    )(hist[:, :n, :], x)

    p = hop(src, dst, y1)

    br = 2000 if n % 2000 == 0 else n
    grid = n // br
    y2 = pl.pallas_call(
        _tc_scale1,
        grid=(grid,),
        in_specs=[
            pl.BlockSpec((NC, br, d), lambda i: (0, i, 0)),
            pl.BlockSpec((br, d), lambda i: (i, 0)),
            pl.BlockSpec((br, 1), lambda i: (i, 0)),
        ],
        out_specs=pl.BlockSpec((br, d), lambda i: (i, 0)),
        out_shape=jax.ShapeDtypeStruct((n, d), jnp.float32),
    )(p[:, :n, :], y1, dinv)

    q = hop(src, dst, y2)

    out = pl.pallas_call(
        _tc_final,
        grid=(grid,),
        in_specs=[
            pl.BlockSpec((NC, br, d), lambda i: (0, i, 0)),
            pl.BlockSpec((br, d), lambda i: (i, 0)),
            pl.BlockSpec((br, 1), lambda i: (i, 0)),
            pl.BlockSpec((d, d), lambda i: (0, 0)),
            pl.BlockSpec((1, d), lambda i: (0, 0)),
        ],
        out_specs=pl.BlockSpec((br, d), lambda i: (i, 0)),
        out_shape=jax.ShapeDtypeStruct((n, d), jnp.float32),
    )(q[:, :n, :], y2, dinv, W.T, b.reshape(1, d))
    return out
